# baseline (speedup 1.0000x reference)
import functools

import jax
import jax.numpy as jnp
from jax import lax
from jax.experimental import pallas as pl
from jax.experimental.pallas import tpu as pltpu
from jax.experimental.pallas import tpu_sc as plsc

_V = 1000000
_D = 64
_B = 16384
_K = 20

_NC, _NS = 2, 16
_NW = _NC * _NS
_BW = _B // _NW
_CB = 32
_NCHUNK = _BW // _CB
_GI = 128
_NEG_I = _CB * _K
_NEG_G = _NEG_I // _GI


def _sc_body(tgt_hbm, ctx_hbm, neg_hbm, emb_hbm, cemb_hbm, pos_hbm, nout_hbm,
             tgt_idx, ctx_idx, neg_idx, tgt_rows, ctx_rows, neg_rows,
             pos_buf, neg_buf, sem):
    wid = lax.axis_index("s") * _NC + lax.axis_index("c")
    base = wid * _BW
    pltpu.sync_copy(tgt_hbm.at[pl.ds(base, _BW)], tgt_idx)
    pltpu.sync_copy(ctx_hbm.at[pl.ds(base, _BW)], ctx_idx)
    pltpu.sync_copy(neg_hbm.at[pl.ds(base * _K, _BW * _K)], neg_idx)
    lane = lax.iota(jnp.int32, 16)

    def chunk_body(c, carry):
        co = c * _CB
        handles = [
            pltpu.async_copy(emb_hbm.at[tgt_idx.at[pl.ds(co, _CB)]],
                             tgt_rows, sem),
            pltpu.async_copy(cemb_hbm.at[ctx_idx.at[pl.ds(co, _CB)]],
                             ctx_rows, sem),
        ]
        for g in range(_NEG_G):
            handles.append(pltpu.async_copy(
                cemb_hbm.at[neg_idx.at[pl.ds(co * _K + g * _GI, _GI)]],
                neg_rows.at[pl.ds(g * _GI, _GI), :], sem))
        for h in handles:
            h.wait()

        for g in range(_CB // 16):
            def item_body(i, accs):
                row = g * 16 + i
                t = [tgt_rows[row, pl.ds(q * 16, 16)] for q in range(4)]
                cx = [ctx_rows[row, pl.ds(q * 16, 16)] for q in range(4)]
                m = lane == i
                acc = (t[0] * cx[0] + t[1] * cx[1]) + (t[2] * cx[2] + t[3] * cx[3])
                out = [jnp.where(m, jnp.sum(acc), accs[0])]
                for k in range(_K):
                    r = row * _K + k
                    n = [neg_rows[r, pl.ds(q * 16, 16)] for q in range(4)]
                    acc = (t[0] * n[0] + t[1] * n[1]) + (t[2] * n[2] + t[3] * n[3])
                    out.append(jnp.where(m, jnp.sum(acc), accs[1 + k]))
                return tuple(out)

            zero = jnp.zeros((16,), jnp.float32)
            accs = lax.fori_loop(0, 16, item_body, (zero,) * (_K + 1))
            pos_buf[pl.ds(co + g * 16, 16)] = accs[0]
            for k in range(_K):
                neg_buf[k, pl.ds(co + g * 16, 16)] = accs[1 + k]
        return carry

    lax.fori_loop(0, _NCHUNK, chunk_body, 0)
    pltpu.sync_copy(pos_buf, pos_hbm.at[pl.ds(base, _BW)])
    pltpu.sync_copy(neg_buf, nout_hbm.at[wid])


@functools.cache
def _sc_dots():
    return pl.kernel(
        _sc_body,
        out_type=(jax.ShapeDtypeStruct((_B,), jnp.float32),
                  jax.ShapeDtypeStruct((_NW, _K, _BW), jnp.float32)),
        mesh=plsc.VectorSubcoreMesh(core_axis_name="c", subcore_axis_name="s",
                                    num_cores=_NC, num_subcores=_NS),
        compiler_params=pltpu.CompilerParams(needs_layout_passes=False,
                                             use_tc_tiling_on_sc=False),
        scratch_types=[
            pltpu.VMEM((_BW,), jnp.int32),
            pltpu.VMEM((_BW,), jnp.int32),
            pltpu.VMEM((_BW * _K,), jnp.int32),
            pltpu.VMEM((_CB, _D), jnp.float32),
            pltpu.VMEM((_CB, _D), jnp.float32),
            pltpu.VMEM((_NEG_I, _D), jnp.float32),
            pltpu.VMEM((_BW,), jnp.float32),
            pltpu.VMEM((_K, _BW), jnp.float32),
            pltpu.SemaphoreType.DMA,
        ],
    )


_TBLK = 8192


def _tp_body(in_ref, out_ref):
    eye = jnp.eye(_D, dtype=jnp.float32)
    out_ref[...] = jax.lax.dot_general(
        in_ref[...], eye, (((0,), (0,)), ((), ())),
        preferred_element_type=jnp.float32)


_tp_call = pl.pallas_call(
    _tp_body,
    grid=(pl.cdiv(_V, _TBLK),),
    in_specs=[pl.BlockSpec((_D, _TBLK), lambda i: (0, i))],
    out_specs=pl.BlockSpec((_TBLK, _D), lambda i: (i, 0)),
    out_shape=jax.ShapeDtypeStruct((_V, _D), jnp.float32),
)


def _loss_body(pos_ref, neg_ref, out_ref):
    pos = pos_ref[...]
    neg = -neg_ref[...]
    ls_pos = jnp.minimum(pos, 0.0) - jnp.log1p(jnp.exp(-jnp.abs(pos)))
    ls_neg = jnp.minimum(neg, 0.0) - jnp.log1p(jnp.exp(-jnp.abs(neg)))
    sp = jnp.sum(ls_pos, axis=0, keepdims=True)
    sn = jnp.sum(ls_neg, axis=0, keepdims=True)
    out_ref[0, 0] = -jnp.sum(sp + sn) / _B


_loss_call = pl.pallas_call(
    _loss_body,
    out_shape=jax.ShapeDtypeStruct((1, 1), jnp.float32),
    out_specs=pl.BlockSpec(memory_space=pltpu.SMEM),
)


def kernel(target, context, negative_samples, emb, ctx_emb):
    tgt = target.astype(jnp.int32)
    ctx = context.astype(jnp.int32)
    neg = negative_samples.astype(jnp.int32).reshape(_B * _K)
    emb_rm = _tp_call(emb.T)
    cemb_rm = _tp_call(ctx_emb.T)
    pos_d, neg_d = _sc_dots()(tgt, ctx, neg, emb_rm, cemb_rm)
    loss = _loss_call(pos_d.reshape(128, 128), neg_d.reshape(_B * _K // 128, 128))
    return loss[0, 0]


# speedup vs baseline: 2.0740x; 2.0740x over previous
import functools

import jax
import jax.numpy as jnp
from jax import lax
from jax.experimental import pallas as pl
from jax.experimental.pallas import tpu as pltpu
from jax.experimental.pallas import tpu_sc as plsc

_V = 1000000
_D = 64
_DP = 128
_B = 16384
_K = 20

_NC, _NS = 2, 16
_NW = _NC * _NS
_BW = _B // _NW
_CB = 32
_NCHUNK = _BW // _CB
_GI = 128
_NEG_I = _CB * _K
_NEG_G = _NEG_I // _GI
_NROW = _K * _BW // _DP
_PROW = _BW // _DP


def _sc_body(tgt_hbm, ctx_hbm, neg_hbm, emb_hbm, cemb_hbm, pos_hbm, nout_hbm,
             tgt_idx, ctx_idx, neg_idx, tgt_rows, ctx_rows, neg_rows,
             pos_buf, neg_buf, sem):
    wid = lax.axis_index("s") * _NC + lax.axis_index("c")
    base = wid * _BW
    pltpu.sync_copy(tgt_hbm.at[pl.ds(base, _BW)], tgt_idx)
    pltpu.sync_copy(ctx_hbm.at[pl.ds(base, _BW)], ctx_idx)
    pltpu.sync_copy(neg_hbm.at[pl.ds(base * _K, _BW * _K)], neg_idx)
    lane = lax.iota(jnp.int32, 16)

    def chunk_body(c, carry):
        co = c * _CB
        handles = [
            pltpu.async_copy(emb_hbm.at[tgt_idx.at[pl.ds(co, _CB)]],
                             tgt_rows, sem),
            pltpu.async_copy(cemb_hbm.at[ctx_idx.at[pl.ds(co, _CB)]],
                             ctx_rows, sem),
        ]
        for g in range(_NEG_G):
            handles.append(pltpu.async_copy(
                cemb_hbm.at[neg_idx.at[pl.ds(co * _K + g * _GI, _GI)]],
                neg_rows.at[pl.ds(g * _GI, _GI), :], sem))
        for h in handles:
            h.wait()

        for g in range(_CB // 16):
            def item_body(i, accs):
                row = g * 16 + i
                t = [tgt_rows[row, pl.ds(q * 16, 16)] for q in range(4)]
                cx = [ctx_rows[row, pl.ds(q * 16, 16)] for q in range(4)]
                m = lane == i
                acc = (t[0] * cx[0] + t[1] * cx[1]) + (t[2] * cx[2] + t[3] * cx[3])
                out = [jnp.where(m, jnp.sum(acc), accs[0])]
                for k in range(_K):
                    r = row * _K + k
                    n = [neg_rows[r, pl.ds(q * 16, 16)] for q in range(4)]
                    acc = (t[0] * n[0] + t[1] * n[1]) + (t[2] * n[2] + t[3] * n[3])
                    out.append(jnp.where(m, jnp.sum(acc), accs[1 + k]))
                return tuple(out)

            zero = jnp.zeros((16,), jnp.float32)
            accs = lax.fori_loop(0, 16, item_body, (zero,) * (_K + 1))
            col = co + g * 16
            pos_buf[col // _DP, pl.ds(col % _DP, 16)] = accs[0]
            for k in range(_K):
                kcol = k * _BW + col
                neg_buf[kcol // _DP, pl.ds(kcol % _DP, 16)] = accs[1 + k]
        return carry

    lax.fori_loop(0, _NCHUNK, chunk_body, 0)
    pltpu.sync_copy(pos_buf, pos_hbm.at[wid])
    pltpu.sync_copy(neg_buf, nout_hbm.at[wid])


@functools.cache
def _sc_dots():
    return pl.kernel(
        _sc_body,
        out_type=(jax.ShapeDtypeStruct((_NW, _PROW, _DP), jnp.float32),
                  jax.ShapeDtypeStruct((_NW, _NROW, _DP), jnp.float32)),
        mesh=plsc.VectorSubcoreMesh(core_axis_name="c", subcore_axis_name="s",
                                    num_cores=_NC, num_subcores=_NS),
        compiler_params=pltpu.CompilerParams(needs_layout_passes=False,
                                             use_tc_tiling_on_sc=True),
        scratch_types=[
            pltpu.VMEM((_BW,), jnp.int32),
            pltpu.VMEM((_BW,), jnp.int32),
            pltpu.VMEM((_BW * _K,), jnp.int32),
            pltpu.VMEM((_CB, _DP), jnp.float32),
            pltpu.VMEM((_CB, _DP), jnp.float32),
            pltpu.VMEM((_NEG_I, _DP), jnp.float32),
            pltpu.VMEM((_PROW, _DP), jnp.float32),
            pltpu.VMEM((_NROW, _DP), jnp.float32),
            pltpu.SemaphoreType.DMA,
        ],
    )


_TBLK = 8192


def _tp_body(in_ref, out_ref):
    eye = jnp.eye(_D, _DP, dtype=jnp.float32)
    out_ref[...] = jax.lax.dot_general(
        in_ref[...], eye, (((0,), (0,)), ((), ())),
        preferred_element_type=jnp.float32)


_tp_call = pl.pallas_call(
    _tp_body,
    grid=(pl.cdiv(_V, _TBLK),),
    in_specs=[pl.BlockSpec((_D, _TBLK), lambda i: (0, i))],
    out_specs=pl.BlockSpec((_TBLK, _DP), lambda i: (i, 0)),
    out_shape=jax.ShapeDtypeStruct((_V, _DP), jnp.float32),
)


def _loss_body(pos_ref, neg_ref, out_ref):
    pos = pos_ref[...]
    neg = -neg_ref[...]
    ls_pos = jnp.minimum(pos, 0.0) - jnp.log1p(jnp.exp(-jnp.abs(pos)))
    ls_neg = jnp.minimum(neg, 0.0) - jnp.log1p(jnp.exp(-jnp.abs(neg)))
    sp = jnp.sum(ls_pos, axis=0, keepdims=True)
    sn = jnp.sum(ls_neg, axis=0, keepdims=True)
    out_ref[0, 0] = -jnp.sum(sp + sn) / _B


_loss_call = pl.pallas_call(
    _loss_body,
    out_shape=jax.ShapeDtypeStruct((1, 1), jnp.float32),
    out_specs=pl.BlockSpec(memory_space=pltpu.SMEM),
)


def kernel(target, context, negative_samples, emb, ctx_emb):
    tgt = target.astype(jnp.int32)
    ctx = context.astype(jnp.int32)
    neg = negative_samples.astype(jnp.int32).reshape(_B * _K)
    emb_rm = _tp_call(emb.T)
    cemb_rm = _tp_call(ctx_emb.T)
    pos_d, neg_d = _sc_dots()(tgt, ctx, neg, emb_rm, cemb_rm)
    loss = _loss_call(pos_d.reshape(_B // _DP, _DP),
                      neg_d.reshape(_B * _K // _DP, _DP))
    return loss[0, 0]


# speedup vs baseline: 2.2458x; 1.0829x over previous
import functools

import jax
import jax.numpy as jnp
from jax import lax
from jax.experimental import pallas as pl
from jax.experimental.pallas import tpu as pltpu
from jax.experimental.pallas import tpu_sc as plsc

_V = 1000000
_D = 64
_DP = 128
_B = 16384
_K = 20

_NC, _NS = 2, 16
_NW = _NC * _NS
_BW = _B // _NW
_CB = 32
_NCHUNK = _BW // _CB
_GI = 128
_NEG_I = _CB * _K
_NEG_G = _NEG_I // _GI
_NROW = _K * _BW // _DP
_PROW = _BW // _DP


def _sc_body(tgt_hbm, ctx_hbm, neg_hbm, emb_hbm, cemb_hbm, pos_hbm, nout_hbm,
             tgt_idx, ctx_idx, neg_idx, tgt_rows, ctx_rows, neg_rows,
             pos_buf, neg_buf, sem):
    wid = lax.axis_index("s") * _NC + lax.axis_index("c")
    base = wid * _BW
    pltpu.sync_copy(tgt_hbm.at[pl.ds(base, _BW)], tgt_idx)
    pltpu.sync_copy(ctx_hbm.at[pl.ds(base, _BW)], ctx_idx)
    pltpu.sync_copy(neg_hbm.at[pl.ds(base * _K, _BW * _K)], neg_idx)
    lane = lax.iota(jnp.int32, 16)

    def chunk_body(c, carry):
        co = c * _CB
        handles = [
            pltpu.async_copy(emb_hbm.at[tgt_idx.at[pl.ds(co, _CB)]],
                             tgt_rows, sem),
            pltpu.async_copy(cemb_hbm.at[ctx_idx.at[pl.ds(co, _CB)]],
                             ctx_rows, sem),
        ]
        for g in range(_NEG_G):
            handles.append(pltpu.async_copy(
                cemb_hbm.at[neg_idx.at[pl.ds(co * _K + g * _GI, _GI)]],
                neg_rows.at[pl.ds(g * _GI, _GI), :], sem))
        for h in handles:
            h.wait()

        for g in range(_CB // 16):
            def item_body(i, accs):
                row = g * 16 + i
                t = [tgt_rows[row, pl.ds(q * 16, 16)] for q in range(4)]
                cx = [ctx_rows[row, pl.ds(q * 16, 16)] for q in range(4)]
                m = lane == i
                acc = (t[0] * cx[0] + t[1] * cx[1]) + (t[2] * cx[2] + t[3] * cx[3])
                out = [jnp.where(m, jnp.sum(acc), accs[0])]
                for k in range(_K):
                    r = row * _K + k
                    n = [neg_rows[r, pl.ds(q * 16, 16)] for q in range(4)]
                    acc = (t[0] * n[0] + t[1] * n[1]) + (t[2] * n[2] + t[3] * n[3])
                    out.append(jnp.where(m, jnp.sum(acc), accs[1 + k]))
                return tuple(out)

            zero = jnp.zeros((16,), jnp.float32)
            accs = lax.fori_loop(0, 16, item_body, (zero,) * (_K + 1))
            col = co + g * 16
            pos_buf[col // _DP, pl.ds(col % _DP, 16)] = accs[0]
            for k in range(_K):
                kcol = k * _BW + col
                neg_buf[kcol // _DP, pl.ds(kcol % _DP, 16)] = accs[1 + k]
        return carry

    lax.fori_loop(0, _NCHUNK, chunk_body, 0)
    pltpu.sync_copy(pos_buf, pos_hbm.at[wid])
    pltpu.sync_copy(neg_buf, nout_hbm.at[wid])


@functools.cache
def _sc_dots():
    return pl.kernel(
        _sc_body,
        out_type=(jax.ShapeDtypeStruct((_NW, _PROW, _DP), jnp.float32),
                  jax.ShapeDtypeStruct((_NW, _NROW, _DP), jnp.float32)),
        mesh=plsc.VectorSubcoreMesh(core_axis_name="c", subcore_axis_name="s",
                                    num_cores=_NC, num_subcores=_NS),
        compiler_params=pltpu.CompilerParams(needs_layout_passes=False,
                                             use_tc_tiling_on_sc=True),
        scratch_types=[
            pltpu.VMEM((_BW,), jnp.int32),
            pltpu.VMEM((_BW,), jnp.int32),
            pltpu.VMEM((_BW * _K,), jnp.int32),
            pltpu.VMEM((_CB, _DP), jnp.float32),
            pltpu.VMEM((_CB, _DP), jnp.float32),
            pltpu.VMEM((_NEG_I, _DP), jnp.float32),
            pltpu.VMEM((_PROW, _DP), jnp.float32),
            pltpu.VMEM((_NROW, _DP), jnp.float32),
            pltpu.SemaphoreType.DMA,
        ],
    )


_TBLK = 16384


def _tp_body(in_ref, out_ref):
    eye = jnp.eye(_D, _DP, dtype=jnp.float32)
    out_ref[...] = jax.lax.dot_general(
        in_ref[...], eye, (((0,), (0,)), ((), ())),
        preferred_element_type=jnp.float32)


_tp_call = pl.pallas_call(
    _tp_body,
    grid=(pl.cdiv(_V, _TBLK),),
    in_specs=[pl.BlockSpec((_D, _TBLK), lambda i: (0, i))],
    out_specs=pl.BlockSpec((_TBLK, _DP), lambda i: (i, 0)),
    out_shape=jax.ShapeDtypeStruct((_V, _DP), jnp.float32),
)


def _loss_body(pos_ref, neg_ref, out_ref):
    pos = pos_ref[...]
    neg = -neg_ref[...]
    ls_pos = jnp.minimum(pos, 0.0) - jnp.log1p(jnp.exp(-jnp.abs(pos)))
    ls_neg = jnp.minimum(neg, 0.0) - jnp.log1p(jnp.exp(-jnp.abs(neg)))
    sp = jnp.sum(ls_pos, axis=0, keepdims=True)
    sn = jnp.sum(ls_neg, axis=0, keepdims=True)
    out_ref[0, 0] = -jnp.sum(sp + sn) / _B


_loss_call = pl.pallas_call(
    _loss_body,
    out_shape=jax.ShapeDtypeStruct((1, 1), jnp.float32),
    out_specs=pl.BlockSpec(memory_space=pltpu.SMEM),
)


def kernel(target, context, negative_samples, emb, ctx_emb):
    tgt = target.astype(jnp.int32)
    ctx = context.astype(jnp.int32)
    neg = negative_samples.astype(jnp.int32).reshape(_B * _K)
    emb_rm = _tp_call(emb.T)
    cemb_rm = _tp_call(ctx_emb.T)
    pos_d, neg_d = _sc_dots()(tgt, ctx, neg, emb_rm, cemb_rm)
    loss = _loss_call(pos_d.reshape(_B // _DP, _DP),
                      neg_d.reshape(_B * _K // _DP, _DP))
    return loss[0, 0]


# speedup vs baseline: 2.2970x; 1.0228x over previous
import functools

import jax
import jax.numpy as jnp
from jax import lax
from jax.experimental import pallas as pl
from jax.experimental.pallas import tpu as pltpu
from jax.experimental.pallas import tpu_sc as plsc

_V = 1000000
_D = 64
_DP = 128
_B = 16384
_K = 20

_NC, _NS = 2, 16
_NW = _NC * _NS
_BW = _B // _NW
_CB = 32
_NCHUNK = _BW // _CB
_GI = 128
_NEG_I = _CB * _K
_NEG_G = _NEG_I // _GI
_NROW = _K * _BW // _DP
_PROW = _BW // _DP


def _sc_body(tgt_hbm, ctx_hbm, neg_hbm, emb_hbm, cemb_hbm, pos_hbm, nout_hbm,
             tgt_idx, ctx_idx, neg_idx, tgt_rows, ctx_rows, neg_rows,
             pos_buf, neg_buf, sem):
    wid = lax.axis_index("s") * _NC + lax.axis_index("c")
    base = wid * _BW
    pltpu.sync_copy(tgt_hbm.at[pl.ds(base, _BW)], tgt_idx)
    pltpu.sync_copy(ctx_hbm.at[pl.ds(base, _BW)], ctx_idx)
    pltpu.sync_copy(neg_hbm.at[pl.ds(base * _K, _BW * _K)], neg_idx)
    lane = lax.iota(jnp.int32, 16)

    def chunk_body(c, carry):
        co = c * _CB
        handles = [
            pltpu.async_copy(emb_hbm.at[tgt_idx.at[pl.ds(co, _CB)]],
                             tgt_rows, sem),
            pltpu.async_copy(cemb_hbm.at[ctx_idx.at[pl.ds(co, _CB)]],
                             ctx_rows, sem),
        ]
        for g in range(_NEG_G):
            handles.append(pltpu.async_copy(
                cemb_hbm.at[neg_idx.at[pl.ds(co * _K + g * _GI, _GI)]],
                neg_rows.at[pl.ds(g * _GI, _GI), :], sem))
        for h in handles:
            h.wait()

        for g in range(_CB // 16):
            def item_body(i, accs):
                row = g * 16 + i
                t = [tgt_rows[row, pl.ds(q * 16, 16)] for q in range(4)]
                cx = [ctx_rows[row, pl.ds(q * 16, 16)] for q in range(4)]
                m = lane == i
                acc = (t[0] * cx[0] + t[1] * cx[1]) + (t[2] * cx[2] + t[3] * cx[3])
                out = [jnp.where(m, jnp.sum(acc), accs[0])]
                for k in range(_K):
                    r = row * _K + k
                    n = [neg_rows[r, pl.ds(q * 16, 16)] for q in range(4)]
                    acc = (t[0] * n[0] + t[1] * n[1]) + (t[2] * n[2] + t[3] * n[3])
                    out.append(jnp.where(m, jnp.sum(acc), accs[1 + k]))
                return tuple(out)

            zero = jnp.zeros((16,), jnp.float32)
            accs = lax.fori_loop(0, 16, item_body, (zero,) * (_K + 1))
            col = co + g * 16
            pos_buf[col // _DP, pl.ds(col % _DP, 16)] = accs[0]
            for k in range(_K):
                kcol = k * _BW + col
                neg_buf[kcol // _DP, pl.ds(kcol % _DP, 16)] = accs[1 + k]
        return carry

    lax.fori_loop(0, _NCHUNK, chunk_body, 0)
    pltpu.sync_copy(pos_buf, pos_hbm.at[wid])
    pltpu.sync_copy(neg_buf, nout_hbm.at[wid])


@functools.cache
def _sc_dots():
    return pl.kernel(
        _sc_body,
        out_type=(jax.ShapeDtypeStruct((_NW, _PROW, _DP), jnp.float32),
                  jax.ShapeDtypeStruct((_NW, _NROW, _DP), jnp.float32)),
        mesh=plsc.VectorSubcoreMesh(core_axis_name="c", subcore_axis_name="s",
                                    num_cores=_NC, num_subcores=_NS),
        compiler_params=pltpu.CompilerParams(needs_layout_passes=False,
                                             use_tc_tiling_on_sc=True),
        scratch_types=[
            pltpu.VMEM((_BW,), jnp.int32),
            pltpu.VMEM((_BW,), jnp.int32),
            pltpu.VMEM((_BW * _K,), jnp.int32),
            pltpu.VMEM((_CB, _DP), jnp.float32),
            pltpu.VMEM((_CB, _DP), jnp.float32),
            pltpu.VMEM((_NEG_I, _DP), jnp.float32),
            pltpu.VMEM((_PROW, _DP), jnp.float32),
            pltpu.VMEM((_NROW, _DP), jnp.float32),
            pltpu.SemaphoreType.DMA,
        ],
    )


_TBLK = 32768


def _tp_body(in_ref, out_ref):
    eye = jnp.eye(_D, _DP, dtype=jnp.float32)
    out_ref[...] = jax.lax.dot_general(
        in_ref[...], eye, (((0,), (0,)), ((), ())),
        preferred_element_type=jnp.float32)


_tp_call = pl.pallas_call(
    _tp_body,
    grid=(pl.cdiv(_V, _TBLK),),
    in_specs=[pl.BlockSpec((_D, _TBLK), lambda i: (0, i))],
    out_specs=pl.BlockSpec((_TBLK, _DP), lambda i: (i, 0)),
    out_shape=jax.ShapeDtypeStruct((_V, _DP), jnp.float32),
)


def _loss_body(pos_ref, neg_ref, out_ref):
    pos = pos_ref[...]
    neg = -neg_ref[...]
    ls_pos = jnp.minimum(pos, 0.0) - jnp.log1p(jnp.exp(-jnp.abs(pos)))
    ls_neg = jnp.minimum(neg, 0.0) - jnp.log1p(jnp.exp(-jnp.abs(neg)))
    sp = jnp.sum(ls_pos, axis=0, keepdims=True)
    sn = jnp.sum(ls_neg, axis=0, keepdims=True)
    out_ref[0, 0] = -jnp.sum(sp + sn) / _B


_loss_call = pl.pallas_call(
    _loss_body,
    out_shape=jax.ShapeDtypeStruct((1, 1), jnp.float32),
    out_specs=pl.BlockSpec(memory_space=pltpu.SMEM),
)


def kernel(target, context, negative_samples, emb, ctx_emb):
    tgt = target.astype(jnp.int32)
    ctx = context.astype(jnp.int32)
    neg = negative_samples.astype(jnp.int32).reshape(_B * _K)
    emb_rm = _tp_call(emb.T)
    cemb_rm = _tp_call(ctx_emb.T)
    pos_d, neg_d = _sc_dots()(tgt, ctx, neg, emb_rm, cemb_rm)
    loss = _loss_call(pos_d.reshape(_B // _DP, _DP),
                      neg_d.reshape(_B * _K // _DP, _DP))
    return loss[0, 0]


# speedup vs baseline: 2.9245x; 1.2732x over previous
import functools

import jax
import jax.numpy as jnp
from jax import lax
from jax.experimental import pallas as pl
from jax.experimental.pallas import tpu as pltpu
from jax.experimental.pallas import tpu_sc as plsc

_V = 1000000
_D = 64
_DP = 128
_B = 16384
_K = 20

_NC, _NS = 2, 16
_NW = _NC * _NS
_BW = _B // _NW
_CB = 32
_NCHUNK = _BW // _CB
_GI = 128
_NEG_I = _CB * _K
_NEG_G = _NEG_I // _GI
_NROW = _K * _BW // _DP
_PROW = _BW // _DP


def _sc_body(tgt_hbm, ctx_hbm, neg_hbm, tab_hbm, pos_hbm, nout_hbm,
             tgt_idx, ctx_idx, neg_idx, tgt_rows, ctx_rows, neg_rows,
             pos_buf, neg_buf, sem):
    wid = lax.axis_index("s") * _NC + lax.axis_index("c")
    base = wid * _BW
    pltpu.sync_copy(tgt_hbm.at[pl.ds(base, _BW)], tgt_idx)
    pltpu.sync_copy(ctx_hbm.at[pl.ds(base, _BW)], ctx_idx)
    pltpu.sync_copy(neg_hbm.at[pl.ds(base * _K, _BW * _K)], neg_idx)
    lane = lax.iota(jnp.int32, 16)

    def chunk_body(c, carry):
        co = c * _CB
        handles = [
            pltpu.async_copy(tab_hbm.at[tgt_idx.at[pl.ds(co, _CB)]],
                             tgt_rows, sem),
            pltpu.async_copy(tab_hbm.at[ctx_idx.at[pl.ds(co, _CB)]],
                             ctx_rows, sem),
        ]
        for g in range(_NEG_G):
            handles.append(pltpu.async_copy(
                tab_hbm.at[neg_idx.at[pl.ds(co * _K + g * _GI, _GI)]],
                neg_rows.at[pl.ds(g * _GI, _GI), :], sem))
        for h in handles:
            h.wait()

        for g in range(_CB // 16):
            def item_body(i, accs):
                row = g * 16 + i
                t = [tgt_rows[row, pl.ds(q * 16, 16)] for q in range(4)]
                cx = [ctx_rows[row, pl.ds(_D + q * 16, 16)] for q in range(4)]
                m = lane == i
                acc = (t[0] * cx[0] + t[1] * cx[1]) + (t[2] * cx[2] + t[3] * cx[3])
                out = [jnp.where(m, jnp.sum(acc), accs[0])]
                for k in range(_K):
                    r = row * _K + k
                    n = [neg_rows[r, pl.ds(_D + q * 16, 16)] for q in range(4)]
                    acc = (t[0] * n[0] + t[1] * n[1]) + (t[2] * n[2] + t[3] * n[3])
                    out.append(jnp.where(m, jnp.sum(acc), accs[1 + k]))
                return tuple(out)

            zero = jnp.zeros((16,), jnp.float32)
            accs = lax.fori_loop(0, 16, item_body, (zero,) * (_K + 1))
            col = co + g * 16
            pos_buf[col // _DP, pl.ds(col % _DP, 16)] = accs[0]
            for k in range(_K):
                kcol = k * _BW + col
                neg_buf[kcol // _DP, pl.ds(kcol % _DP, 16)] = accs[1 + k]
        return carry

    lax.fori_loop(0, _NCHUNK, chunk_body, 0)
    pltpu.sync_copy(pos_buf, pos_hbm.at[wid])
    pltpu.sync_copy(neg_buf, nout_hbm.at[wid])


@functools.cache
def _sc_dots():
    return pl.kernel(
        _sc_body,
        out_type=(jax.ShapeDtypeStruct((_NW, _PROW, _DP), jnp.float32),
                  jax.ShapeDtypeStruct((_NW, _NROW, _DP), jnp.float32)),
        mesh=plsc.VectorSubcoreMesh(core_axis_name="c", subcore_axis_name="s",
                                    num_cores=_NC, num_subcores=_NS),
        compiler_params=pltpu.CompilerParams(needs_layout_passes=False,
                                             use_tc_tiling_on_sc=True),
        scratch_types=[
            pltpu.VMEM((_BW,), jnp.int32),
            pltpu.VMEM((_BW,), jnp.int32),
            pltpu.VMEM((_BW * _K,), jnp.int32),
            pltpu.VMEM((_CB, _DP), jnp.float32),
            pltpu.VMEM((_CB, _DP), jnp.float32),
            pltpu.VMEM((_NEG_I, _DP), jnp.float32),
            pltpu.VMEM((_PROW, _DP), jnp.float32),
            pltpu.VMEM((_NROW, _DP), jnp.float32),
            pltpu.SemaphoreType.DMA,
        ],
    )


_TBLK = 16384


def _tp_body(emb_ref, cemb_ref, out_ref):
    eye_lo = jnp.eye(_D, _DP, dtype=jnp.float32)
    eye_hi = jnp.concatenate(
        [jnp.zeros((_D, _D), jnp.float32), jnp.eye(_D, dtype=jnp.float32)],
        axis=1)
    dn = (((0,), (0,)), ((), ()))
    out_ref[...] = (
        jax.lax.dot_general(emb_ref[...], eye_lo, dn,
                            preferred_element_type=jnp.float32)
        + jax.lax.dot_general(cemb_ref[...], eye_hi, dn,
                              preferred_element_type=jnp.float32))


_tp_call = pl.pallas_call(
    _tp_body,
    grid=(pl.cdiv(_V, _TBLK),),
    in_specs=[pl.BlockSpec((_D, _TBLK), lambda i: (0, i)),
              pl.BlockSpec((_D, _TBLK), lambda i: (0, i))],
    out_specs=pl.BlockSpec((_TBLK, _DP), lambda i: (i, 0)),
    out_shape=jax.ShapeDtypeStruct((_V, _DP), jnp.float32),
)


def _loss_body(pos_ref, neg_ref, out_ref):
    pos = pos_ref[...]
    neg = -neg_ref[...]
    ls_pos = jnp.minimum(pos, 0.0) - jnp.log1p(jnp.exp(-jnp.abs(pos)))
    ls_neg = jnp.minimum(neg, 0.0) - jnp.log1p(jnp.exp(-jnp.abs(neg)))
    sp = jnp.sum(ls_pos, axis=0, keepdims=True)
    sn = jnp.sum(ls_neg, axis=0, keepdims=True)
    out_ref[0, 0] = -jnp.sum(sp + sn) / _B


_loss_call = pl.pallas_call(
    _loss_body,
    out_shape=jax.ShapeDtypeStruct((1, 1), jnp.float32),
    out_specs=pl.BlockSpec(memory_space=pltpu.SMEM),
)


def kernel(target, context, negative_samples, emb, ctx_emb):
    tgt = target.astype(jnp.int32)
    ctx = context.astype(jnp.int32)
    neg = negative_samples.astype(jnp.int32).reshape(_B * _K)
    comb = _tp_call(emb.T, ctx_emb.T)
    pos_d, neg_d = _sc_dots()(tgt, ctx, neg, comb)
    loss = _loss_call(pos_d.reshape(_B // _DP, _DP),
                      neg_d.reshape(_B * _K // _DP, _DP))
    return loss[0, 0]


# speedup vs baseline: 3.1182x; 1.0662x over previous
import functools

import jax
import jax.numpy as jnp
from jax import lax
from jax.experimental import pallas as pl
from jax.experimental.pallas import tpu as pltpu
from jax.experimental.pallas import tpu_sc as plsc

_V = 1000000
_D = 64
_DP = 128
_B = 16384
_K = 20

_NC, _NS = 2, 16
_NW = _NC * _NS
_BW = _B // _NW
_CB = 32
_NCHUNK = _BW // _CB
_GI = 128
_NEG_I = _CB * _K
_NEG_G = _NEG_I // _GI
_NROW = _K * _BW // _DP
_PROW = _BW // _DP


def _sc_body(tgt_hbm, ctx_hbm, neg_hbm, tab_hbm, pos_hbm, nout_hbm,
             tgt_idx, ctx_idx, neg_idx, tgt_rows, ctx_rows, neg_rows,
             pos_buf, neg_buf, sem):
    wid = lax.axis_index("s") * _NC + lax.axis_index("c")
    base = wid * _BW
    pltpu.sync_copy(tgt_hbm.at[pl.ds(base, _BW)], tgt_idx)
    pltpu.sync_copy(ctx_hbm.at[pl.ds(base, _BW)], ctx_idx)
    pltpu.sync_copy(neg_hbm.at[pl.ds(base * _K, _BW * _K)], neg_idx)
    lane = lax.iota(jnp.int32, 16)

    def chunk_body(c, carry):
        co = c * _CB
        handles = [
            pltpu.async_copy(tab_hbm.at[tgt_idx.at[pl.ds(co, _CB)]],
                             tgt_rows, sem),
            pltpu.async_copy(tab_hbm.at[ctx_idx.at[pl.ds(co, _CB)]],
                             ctx_rows, sem),
        ]
        for g in range(_NEG_G):
            handles.append(pltpu.async_copy(
                tab_hbm.at[neg_idx.at[pl.ds(co * _K + g * _GI, _GI)]],
                neg_rows.at[pl.ds(g * _GI, _GI), :], sem))
        for h in handles:
            h.wait()

        for g in range(_CB // 16):
            def item_body(i, accs):
                row = g * 16 + i
                t = [tgt_rows[row, pl.ds(q * 16, 16)] for q in range(4)]
                cx = [ctx_rows[row, pl.ds(q * 16, 16)] for q in range(4)]
                m = lane == i
                acc = (t[0] * cx[0] + t[1] * cx[1]) + (t[2] * cx[2] + t[3] * cx[3])
                out = [jnp.where(m, jnp.sum(acc), accs[0])]
                for k in range(_K):
                    r = row * _K + k
                    n = [neg_rows[r, pl.ds(q * 16, 16)] for q in range(4)]
                    acc = (t[0] * n[0] + t[1] * n[1]) + (t[2] * n[2] + t[3] * n[3])
                    out.append(jnp.where(m, jnp.sum(acc), accs[1 + k]))
                return tuple(out)

            zero = jnp.zeros((16,), jnp.float32)
            accs = lax.fori_loop(0, 16, item_body, (zero,) * (_K + 1))
            col = co + g * 16
            pos_buf[col // _DP, pl.ds(col % _DP, 16)] = accs[0]
            for k in range(_K):
                kcol = k * _BW + col
                neg_buf[kcol // _DP, pl.ds(kcol % _DP, 16)] = accs[1 + k]
        return carry

    lax.fori_loop(0, _NCHUNK, chunk_body, 0)
    pltpu.sync_copy(pos_buf, pos_hbm.at[wid])
    pltpu.sync_copy(neg_buf, nout_hbm.at[wid])


@functools.cache
def _sc_dots():
    return pl.kernel(
        _sc_body,
        out_type=(jax.ShapeDtypeStruct((_NW, _PROW, _DP), jnp.float32),
                  jax.ShapeDtypeStruct((_NW, _NROW, _DP), jnp.float32)),
        mesh=plsc.VectorSubcoreMesh(core_axis_name="c", subcore_axis_name="s",
                                    num_cores=_NC, num_subcores=_NS),
        compiler_params=pltpu.CompilerParams(needs_layout_passes=False,
                                             use_tc_tiling_on_sc=False),
        scratch_types=[
            pltpu.VMEM((_BW,), jnp.int32),
            pltpu.VMEM((_BW,), jnp.int32),
            pltpu.VMEM((_BW * _K,), jnp.int32),
            pltpu.VMEM((_CB, _D), jnp.float32),
            pltpu.VMEM((_CB, _D), jnp.float32),
            pltpu.VMEM((_NEG_I, _D), jnp.float32),
            pltpu.VMEM((_PROW, _DP), jnp.float32),
            pltpu.VMEM((_NROW, _DP), jnp.float32),
            pltpu.SemaphoreType.DMA,
        ],
    )


_TBLK = 16384


def _tp_body(emb_ref, cemb_ref, out_ref):
    eye_lo = jnp.eye(_D, _DP, dtype=jnp.float32)
    eye_hi = jnp.concatenate(
        [jnp.zeros((_D, _D), jnp.float32), jnp.eye(_D, dtype=jnp.float32)],
        axis=1)
    dn = (((0,), (0,)), ((), ()))
    out_ref[...] = (
        jax.lax.dot_general(emb_ref[...], eye_lo, dn,
                            preferred_element_type=jnp.float32)
        + jax.lax.dot_general(cemb_ref[...], eye_hi, dn,
                              preferred_element_type=jnp.float32))


_tp_call = pl.pallas_call(
    _tp_body,
    grid=(pl.cdiv(_V, _TBLK),),
    in_specs=[pl.BlockSpec((_D, _TBLK), lambda i: (0, i)),
              pl.BlockSpec((_D, _TBLK), lambda i: (0, i))],
    out_specs=pl.BlockSpec((_TBLK, _DP), lambda i: (i, 0)),
    out_shape=jax.ShapeDtypeStruct((_V, _DP), jnp.float32),
)


def _loss_body(pos_ref, neg_ref, out_ref):
    pos = pos_ref[...]
    neg = -neg_ref[...]
    ls_pos = jnp.minimum(pos, 0.0) - jnp.log1p(jnp.exp(-jnp.abs(pos)))
    ls_neg = jnp.minimum(neg, 0.0) - jnp.log1p(jnp.exp(-jnp.abs(neg)))
    sp = jnp.sum(ls_pos, axis=0, keepdims=True)
    sn = jnp.sum(ls_neg, axis=0, keepdims=True)
    out_ref[0, 0] = -jnp.sum(sp + sn) / _B


_loss_call = pl.pallas_call(
    _loss_body,
    out_shape=jax.ShapeDtypeStruct((1, 1), jnp.float32),
    out_specs=pl.BlockSpec(memory_space=pltpu.SMEM),
)


def kernel(target, context, negative_samples, emb, ctx_emb):
    tgt = target.astype(jnp.int32)
    ctx = context.astype(jnp.int32)
    neg = negative_samples.astype(jnp.int32).reshape(_B * _K)
    comb = _tp_call(emb.T, ctx_emb.T)
    comb2 = comb.reshape(2 * _V, _D)
    pos_d, neg_d = _sc_dots()(tgt * 2, ctx * 2 + 1, neg * 2 + 1, comb2)
    loss = _loss_call(pos_d.reshape(_B // _DP, _DP),
                      neg_d.reshape(_B * _K // _DP, _DP))
    return loss[0, 0]


# speedup vs baseline: 3.1982x; 1.0257x over previous
import functools

import jax
import jax.numpy as jnp
from jax import lax
from jax.experimental import pallas as pl
from jax.experimental.pallas import tpu as pltpu
from jax.experimental.pallas import tpu_sc as plsc

_V = 1000000
_D = 64
_DP = 128
_B = 16384
_K = 20

_NC, _NS = 2, 16
_NW = _NC * _NS
_BW = _B // _NW
_CB = 32
_NCHUNK = _BW // _CB
_GI = 128
_NEG_I = _CB * _K
_NEG_G = _NEG_I // _GI
_NROW = _K * _BW // _DP
_PROW = _BW // _DP


def _sc_body(tgt_hbm, ctx_hbm, neg_hbm, tab_hbm, pos_hbm, nout_hbm,
             tgt_idx, ctx_idx, neg_idx, tgt_rows, ctx_rows, neg_rows,
             pos_buf, neg_buf, sem):
    wid = lax.axis_index("s") * _NC + lax.axis_index("c")
    base = wid * _BW
    pltpu.sync_copy(tgt_hbm.at[pl.ds(base, _BW)], tgt_idx)
    pltpu.sync_copy(ctx_hbm.at[pl.ds(base, _BW)], ctx_idx)
    pltpu.sync_copy(neg_hbm.at[pl.ds(base * _K, _BW * _K)], neg_idx)
    lane = lax.iota(jnp.int32, 16)

    def chunk_body(c, carry):
        co = c * _CB
        handles = [
            pltpu.async_copy(tab_hbm.at[tgt_idx.at[pl.ds(co, _CB)]],
                             tgt_rows, sem),
            pltpu.async_copy(tab_hbm.at[ctx_idx.at[pl.ds(co, _CB)]],
                             ctx_rows, sem),
        ]
        for g in range(_NEG_G):
            handles.append(pltpu.async_copy(
                tab_hbm.at[neg_idx.at[pl.ds(co * _K + g * _GI, _GI)]],
                neg_rows.at[pl.ds(g * _GI, _GI), :], sem))
        for h in handles:
            h.wait()

        for g in range(_CB // 16):
            def item_body(i, accs):
                row = g * 16 + i
                t = [tgt_rows[row, pl.ds(q * 16, 16)] for q in range(4)]
                cx = [ctx_rows[row, pl.ds(q * 16, 16)] for q in range(4)]
                m = lane == i
                acc = (t[0] * cx[0] + t[1] * cx[1]) + (t[2] * cx[2] + t[3] * cx[3])
                out = [jnp.where(m, jnp.sum(acc), accs[0])]
                for k in range(_K):
                    r = row * _K + k
                    n = [neg_rows[r, pl.ds(q * 16, 16)] for q in range(4)]
                    acc = (t[0] * n[0] + t[1] * n[1]) + (t[2] * n[2] + t[3] * n[3])
                    out.append(jnp.where(m, jnp.sum(acc), accs[1 + k]))
                return tuple(out)

            zero = jnp.zeros((16,), jnp.float32)
            accs = lax.fori_loop(0, 16, item_body, (zero,) * (_K + 1))
            col = co + g * 16
            pos_buf[col // _DP, pl.ds(col % _DP, 16)] = accs[0]
            for k in range(_K):
                kcol = k * _BW + col
                neg_buf[kcol // _DP, pl.ds(kcol % _DP, 16)] = accs[1 + k]
        return carry

    lax.fori_loop(0, _NCHUNK, chunk_body, 0)
    pltpu.sync_copy(pos_buf, pos_hbm.at[wid])
    pltpu.sync_copy(neg_buf, nout_hbm.at[wid])


@functools.cache
def _sc_dots():
    return pl.kernel(
        _sc_body,
        out_type=(jax.ShapeDtypeStruct((_NW, _PROW, _DP), jnp.float32),
                  jax.ShapeDtypeStruct((_NW, _NROW, _DP), jnp.float32)),
        mesh=plsc.VectorSubcoreMesh(core_axis_name="c", subcore_axis_name="s",
                                    num_cores=_NC, num_subcores=_NS),
        compiler_params=pltpu.CompilerParams(needs_layout_passes=False,
                                             use_tc_tiling_on_sc=False),
        scratch_types=[
            pltpu.VMEM((_BW,), jnp.int32),
            pltpu.VMEM((_BW,), jnp.int32),
            pltpu.VMEM((_BW * _K,), jnp.int32),
            pltpu.VMEM((_CB, _D), jnp.float32),
            pltpu.VMEM((_CB, _D), jnp.float32),
            pltpu.VMEM((_NEG_I, _D), jnp.float32),
            pltpu.VMEM((_PROW, _DP), jnp.float32),
            pltpu.VMEM((_NROW, _DP), jnp.float32),
            pltpu.SemaphoreType.DMA,
        ],
    )


_TBLK = 20480


def _tp_body(emb_ref, cemb_ref, out_ref):
    eye_lo = jnp.eye(_D, _DP, dtype=jnp.float32)
    eye_hi = jnp.concatenate(
        [jnp.zeros((_D, _D), jnp.float32), jnp.eye(_D, dtype=jnp.float32)],
        axis=1)
    dn = (((0,), (0,)), ((), ()))
    out_ref[...] = (
        jax.lax.dot_general(emb_ref[...], eye_lo, dn,
                            preferred_element_type=jnp.float32)
        + jax.lax.dot_general(cemb_ref[...], eye_hi, dn,
                              preferred_element_type=jnp.float32))


_tp_call = pl.pallas_call(
    _tp_body,
    grid=(pl.cdiv(_V, _TBLK),),
    in_specs=[pl.BlockSpec((_D, _TBLK), lambda i: (0, i)),
              pl.BlockSpec((_D, _TBLK), lambda i: (0, i))],
    out_specs=pl.BlockSpec((_TBLK, _DP), lambda i: (i, 0)),
    out_shape=jax.ShapeDtypeStruct((_V, _DP), jnp.float32),
)


def _loss_body(pos_ref, neg_ref, out_ref):
    pos = pos_ref[...]
    neg = -neg_ref[...]
    ls_pos = jnp.minimum(pos, 0.0) - jnp.log1p(jnp.exp(-jnp.abs(pos)))
    ls_neg = jnp.minimum(neg, 0.0) - jnp.log1p(jnp.exp(-jnp.abs(neg)))
    sp = jnp.sum(ls_pos, axis=0, keepdims=True)
    sn = jnp.sum(ls_neg, axis=0, keepdims=True)
    out_ref[0, 0] = -jnp.sum(sp + sn) / _B


_loss_call = pl.pallas_call(
    _loss_body,
    out_shape=jax.ShapeDtypeStruct((1, 1), jnp.float32),
    out_specs=pl.BlockSpec(memory_space=pltpu.SMEM),
)


def kernel(target, context, negative_samples, emb, ctx_emb):
    tgt = target.astype(jnp.int32)
    ctx = context.astype(jnp.int32)
    neg = negative_samples.astype(jnp.int32).reshape(_B * _K)
    comb = _tp_call(emb.T, ctx_emb.T)
    comb2 = comb.reshape(2 * _V, _D)
    pos_d, neg_d = _sc_dots()(tgt * 2, ctx * 2 + 1, neg * 2 + 1, comb2)
    loss = _loss_call(pos_d.reshape(_B // _DP, _DP),
                      neg_d.reshape(_B * _K // _DP, _DP))
    return loss[0, 0]


# speedup vs baseline: 3.4165x; 1.0683x over previous
import functools

import jax
import jax.numpy as jnp
from jax import lax
from jax.experimental import pallas as pl
from jax.experimental.pallas import tpu as pltpu
from jax.experimental.pallas import tpu_sc as plsc

_V = 1000000
_D = 64
_DP = 128
_B = 16384
_K = 20

_NC, _NS = 2, 16
_NW = _NC * _NS
_BW = _B // _NW
_CB = 32
_NCHUNK = _BW // _CB
_GI = 128
_NEG_I = _CB * _K
_NEG_G = _NEG_I // _GI
_NROW = _K * _BW // _DP
_PROW = _BW // _DP


def _sc_body(tgt_hbm, ctx_hbm, neg_hbm, tab_hbm, pos_hbm, nout_hbm,
             tgt_idx, ctx_idx, neg_idx,
             tgt_rows0, ctx_rows0, neg_rows0,
             tgt_rows1, ctx_rows1, neg_rows1,
             pos_buf, neg_buf, sem0, sem1):
    wid = lax.axis_index("s") * _NC + lax.axis_index("c")
    base = wid * _BW
    pltpu.sync_copy(tgt_hbm.at[pl.ds(base, _BW)], tgt_idx)
    pltpu.sync_copy(ctx_hbm.at[pl.ds(base, _BW)], ctx_idx)
    pltpu.sync_copy(neg_hbm.at[pl.ds(base * _K, _BW * _K)], neg_idx)
    lane = lax.iota(jnp.int32, 16)

    bufs = ((tgt_rows0, ctx_rows0, neg_rows0, sem0),
            (tgt_rows1, ctx_rows1, neg_rows1, sem1))

    def issue(c, b):
        tr, cr, nr, sem = bufs[b]
        co = c * _CB
        pltpu.async_copy(tab_hbm.at[tgt_idx.at[pl.ds(co, _CB)]], tr, sem)
        pltpu.async_copy(tab_hbm.at[ctx_idx.at[pl.ds(co, _CB)]], cr, sem)
        for g in range(_NEG_G):
            pltpu.async_copy(
                tab_hbm.at[neg_idx.at[pl.ds(co * _K + g * _GI, _GI)]],
                nr.at[pl.ds(g * _GI, _GI), :], sem)

    def wait(b):
        tr, cr, nr, sem = bufs[b]
        pltpu.make_async_copy(tab_hbm.at[pl.ds(0, _CB), :], tr, sem).wait()
        pltpu.make_async_copy(tab_hbm.at[pl.ds(0, _CB), :], cr, sem).wait()
        pltpu.make_async_copy(tab_hbm.at[pl.ds(0, _NEG_I), :], nr, sem).wait()

    def compute(c, b):
        tr, cr, nr, _ = bufs[b]
        co = c * _CB
        for g in range(_CB // 16):
            def item_body(i, accs):
                row = g * 16 + i
                t = [tr[row, pl.ds(q * 16, 16)] for q in range(4)]
                cx = [cr[row, pl.ds(q * 16, 16)] for q in range(4)]
                m = lane == i
                acc = (t[0] * cx[0] + t[1] * cx[1]) + (t[2] * cx[2] + t[3] * cx[3])
                out = [jnp.where(m, jnp.sum(acc), accs[0])]
                for k in range(_K):
                    r = row * _K + k
                    n = [nr[r, pl.ds(q * 16, 16)] for q in range(4)]
                    acc = (t[0] * n[0] + t[1] * n[1]) + (t[2] * n[2] + t[3] * n[3])
                    out.append(jnp.where(m, jnp.sum(acc), accs[1 + k]))
                return tuple(out)

            zero = jnp.zeros((16,), jnp.float32)
            accs = lax.fori_loop(0, 16, item_body, (zero,) * (_K + 1))
            col = co + g * 16
            pos_buf[col // _DP, pl.ds(col % _DP, 16)] = accs[0]
            for k in range(_K):
                kcol = k * _BW + col
                neg_buf[kcol // _DP, pl.ds(kcol % _DP, 16)] = accs[1 + k]

    issue(0, 0)
    issue(1, 1)

    def pair_body(j, carry):
        c0 = 2 * j
        for b in range(2):
            c = c0 + b
            wait(b)
            compute(c, b)

            @pl.when(c + 2 < _NCHUNK)
            def _():
                issue(c + 2, b)
        return carry

    lax.fori_loop(0, _NCHUNK // 2, pair_body, 0)
    pltpu.sync_copy(pos_buf, pos_hbm.at[wid])
    pltpu.sync_copy(neg_buf, nout_hbm.at[wid])


@functools.cache
def _sc_dots():
    return pl.kernel(
        _sc_body,
        out_type=(jax.ShapeDtypeStruct((_NW, _PROW, _DP), jnp.float32),
                  jax.ShapeDtypeStruct((_NW, _NROW, _DP), jnp.float32)),
        mesh=plsc.VectorSubcoreMesh(core_axis_name="c", subcore_axis_name="s",
                                    num_cores=_NC, num_subcores=_NS),
        compiler_params=pltpu.CompilerParams(needs_layout_passes=False,
                                             use_tc_tiling_on_sc=False),
        scratch_types=[
            pltpu.VMEM((_BW,), jnp.int32),
            pltpu.VMEM((_BW,), jnp.int32),
            pltpu.VMEM((_BW * _K,), jnp.int32),
            pltpu.VMEM((_CB, _D), jnp.float32),
            pltpu.VMEM((_CB, _D), jnp.float32),
            pltpu.VMEM((_NEG_I, _D), jnp.float32),
            pltpu.VMEM((_CB, _D), jnp.float32),
            pltpu.VMEM((_CB, _D), jnp.float32),
            pltpu.VMEM((_NEG_I, _D), jnp.float32),
            pltpu.VMEM((_PROW, _DP), jnp.float32),
            pltpu.VMEM((_NROW, _DP), jnp.float32),
            pltpu.SemaphoreType.DMA,
            pltpu.SemaphoreType.DMA,
        ],
    )


_TBLK = 20480


def _tp_body(emb_ref, cemb_ref, out_ref):
    eye_lo = jnp.eye(_D, _DP, dtype=jnp.float32)
    eye_hi = jnp.concatenate(
        [jnp.zeros((_D, _D), jnp.float32), jnp.eye(_D, dtype=jnp.float32)],
        axis=1)
    dn = (((0,), (0,)), ((), ()))
    out_ref[...] = (
        jax.lax.dot_general(emb_ref[...], eye_lo, dn,
                            preferred_element_type=jnp.float32)
        + jax.lax.dot_general(cemb_ref[...], eye_hi, dn,
                              preferred_element_type=jnp.float32))


_tp_call = pl.pallas_call(
    _tp_body,
    grid=(pl.cdiv(_V, _TBLK),),
    in_specs=[pl.BlockSpec((_D, _TBLK), lambda i: (0, i)),
              pl.BlockSpec((_D, _TBLK), lambda i: (0, i))],
    out_specs=pl.BlockSpec((_TBLK, _DP), lambda i: (i, 0)),
    out_shape=jax.ShapeDtypeStruct((_V, _DP), jnp.float32),
)


def _loss_body(pos_ref, neg_ref, out_ref):
    pos = pos_ref[...]
    neg = -neg_ref[...]
    ls_pos = jnp.minimum(pos, 0.0) - jnp.log1p(jnp.exp(-jnp.abs(pos)))
    ls_neg = jnp.minimum(neg, 0.0) - jnp.log1p(jnp.exp(-jnp.abs(neg)))
    sp = jnp.sum(ls_pos, axis=0, keepdims=True)
    sn = jnp.sum(ls_neg, axis=0, keepdims=True)
    out_ref[0, 0] = -jnp.sum(sp + sn) / _B


_loss_call = pl.pallas_call(
    _loss_body,
    out_shape=jax.ShapeDtypeStruct((1, 1), jnp.float32),
    out_specs=pl.BlockSpec(memory_space=pltpu.SMEM),
)


def kernel(target, context, negative_samples, emb, ctx_emb):
    tgt = target.astype(jnp.int32)
    ctx = context.astype(jnp.int32)
    neg = negative_samples.astype(jnp.int32).reshape(_B * _K)
    comb = _tp_call(emb.T, ctx_emb.T)
    comb2 = comb.reshape(2 * _V, _D)
    pos_d, neg_d = _sc_dots()(tgt * 2, ctx * 2 + 1, neg * 2 + 1, comb2)
    loss = _loss_call(pos_d.reshape(_B // _DP, _DP),
                      neg_d.reshape(_B * _K // _DP, _DP))
    return loss[0, 0]


# speedup vs baseline: 3.5431x; 1.0371x over previous
import functools

import jax
import jax.numpy as jnp
from jax import lax
from jax.experimental import pallas as pl
from jax.experimental.pallas import tpu as pltpu
from jax.experimental.pallas import tpu_sc as plsc

_V = 1000000
_D = 64
_DP = 128
_B = 16384
_K = 20

_NC, _NS = 2, 16
_NW = _NC * _NS
_BW = _B // _NW
_CB = 32
_NCHUNK = _BW // _CB
_GI = 128
_NEG_I = _CB * _K
_NEG_G = _NEG_I // _GI
_NROW = _K * _BW // _DP
_PROW = _BW // _DP


def _sc_body(tgt_hbm, ctx_hbm, neg_hbm, tab_hbm, pos_hbm, nout_hbm,
             tgt_idx, ctx_idx, neg_idx,
             tgt_rows0, ctx_rows0, neg_rows0,
             tgt_rows1, ctx_rows1, neg_rows1,
             pos_buf, neg_buf, sem0, sem1):
    wid = lax.axis_index("s") * _NC + lax.axis_index("c")
    base = wid * _BW
    pltpu.sync_copy(tgt_hbm.at[pl.ds(base, _BW)], tgt_idx)
    pltpu.sync_copy(ctx_hbm.at[pl.ds(base, _BW)], ctx_idx)
    pltpu.sync_copy(neg_hbm.at[pl.ds(base * _K, _BW * _K)], neg_idx)
    lane = lax.iota(jnp.int32, 16)

    bufs = ((tgt_rows0, ctx_rows0, neg_rows0, sem0),
            (tgt_rows1, ctx_rows1, neg_rows1, sem1))

    def issue(c, b):
        tr, cr, nr, sem = bufs[b]
        co = c * _CB
        pltpu.async_copy(tab_hbm.at[tgt_idx.at[pl.ds(co, _CB)]], tr, sem)
        pltpu.async_copy(tab_hbm.at[ctx_idx.at[pl.ds(co, _CB)]], cr, sem)
        for g in range(_NEG_G):
            pltpu.async_copy(
                tab_hbm.at[neg_idx.at[pl.ds(co * _K + g * _GI, _GI)]],
                nr.at[pl.ds(g * _GI, _GI), :], sem)

    def wait(b):
        tr, cr, nr, sem = bufs[b]
        pltpu.make_async_copy(tab_hbm.at[pl.ds(0, _CB), :], tr, sem).wait()
        pltpu.make_async_copy(tab_hbm.at[pl.ds(0, _CB), :], cr, sem).wait()
        pltpu.make_async_copy(tab_hbm.at[pl.ds(0, _NEG_I), :], nr, sem).wait()

    def compute(c, b):
        tr, cr, nr, _ = bufs[b]
        co = c * _CB
        for g in range(_CB // 16):
            def item_body(i, accs):
                row = g * 16 + i
                t = [tr[row, pl.ds(q * 16, 16)] for q in range(4)]
                cx = [cr[row, pl.ds(q * 16, 16)] for q in range(4)]
                m = lane == i
                acc = (t[0] * cx[0] + t[1] * cx[1]) + (t[2] * cx[2] + t[3] * cx[3])
                out = [jnp.where(m, jnp.sum(acc), accs[0])]
                for k in range(_K):
                    r = row * _K + k
                    n = [nr[r, pl.ds(q * 16, 16)] for q in range(4)]
                    acc = (t[0] * n[0] + t[1] * n[1]) + (t[2] * n[2] + t[3] * n[3])
                    out.append(jnp.where(m, jnp.sum(acc), accs[1 + k]))
                return tuple(out)

            zero = jnp.zeros((16,), jnp.float32)
            accs = lax.fori_loop(0, 16, item_body, (zero,) * (_K + 1))
            col = co + g * 16
            pos_buf[col // _DP, pl.ds(col % _DP, 16)] = accs[0]
            for k in range(_K):
                kcol = k * _BW + col
                neg_buf[kcol // _DP, pl.ds(kcol % _DP, 16)] = accs[1 + k]

    issue(0, 0)
    issue(1, 1)

    def pair_body(j, carry):
        c0 = 2 * j
        for b in range(2):
            c = c0 + b
            wait(b)
            compute(c, b)

            @pl.when(c + 2 < _NCHUNK)
            def _():
                issue(c + 2, b)
        return carry

    lax.fori_loop(0, _NCHUNK // 2, pair_body, 0)
    pltpu.sync_copy(pos_buf, pos_hbm.at[wid])
    pltpu.sync_copy(neg_buf, nout_hbm.at[wid])


@functools.cache
def _sc_dots():
    return pl.kernel(
        _sc_body,
        out_type=(jax.ShapeDtypeStruct((_NW, _PROW, _DP), jnp.float32),
                  jax.ShapeDtypeStruct((_NW, _NROW, _DP), jnp.float32)),
        mesh=plsc.VectorSubcoreMesh(core_axis_name="c", subcore_axis_name="s",
                                    num_cores=_NC, num_subcores=_NS),
        compiler_params=pltpu.CompilerParams(needs_layout_passes=False,
                                             use_tc_tiling_on_sc=False),
        scratch_types=[
            pltpu.VMEM((_BW,), jnp.int32),
            pltpu.VMEM((_BW,), jnp.int32),
            pltpu.VMEM((_BW * _K,), jnp.int32),
            pltpu.VMEM((_CB, _D), jnp.float32),
            pltpu.VMEM((_CB, _D), jnp.float32),
            pltpu.VMEM((_NEG_I, _D), jnp.float32),
            pltpu.VMEM((_CB, _D), jnp.float32),
            pltpu.VMEM((_CB, _D), jnp.float32),
            pltpu.VMEM((_NEG_I, _D), jnp.float32),
            pltpu.VMEM((_PROW, _DP), jnp.float32),
            pltpu.VMEM((_NROW, _DP), jnp.float32),
            pltpu.SemaphoreType.DMA,
            pltpu.SemaphoreType.DMA,
        ],
    )


_TBLK = 20480


def _tp_body(emb_ref, cemb_ref, out_ref):
    eye = jnp.eye(_DP, dtype=jnp.float32)
    dn = (((0,), (0,)), ((), ()))
    x = jnp.concatenate([emb_ref[...], cemb_ref[...]], axis=0)
    out_ref[...] = jax.lax.dot_general(x, eye, dn,
                                       preferred_element_type=jnp.float32)


_tp_call = pl.pallas_call(
    _tp_body,
    grid=(pl.cdiv(_V, _TBLK),),
    in_specs=[pl.BlockSpec((_D, _TBLK), lambda i: (0, i)),
              pl.BlockSpec((_D, _TBLK), lambda i: (0, i))],
    out_specs=pl.BlockSpec((_TBLK, _DP), lambda i: (i, 0)),
    out_shape=jax.ShapeDtypeStruct((_V, _DP), jnp.float32),
)


def _loss_body(pos_ref, neg_ref, out_ref):
    pos = pos_ref[...]
    neg = -neg_ref[...]
    ls_pos = jnp.minimum(pos, 0.0) - jnp.log1p(jnp.exp(-jnp.abs(pos)))
    ls_neg = jnp.minimum(neg, 0.0) - jnp.log1p(jnp.exp(-jnp.abs(neg)))
    sp = jnp.sum(ls_pos, axis=0, keepdims=True)
    sn = jnp.sum(ls_neg, axis=0, keepdims=True)
    out_ref[0, 0] = -jnp.sum(sp + sn) / _B


_loss_call = pl.pallas_call(
    _loss_body,
    out_shape=jax.ShapeDtypeStruct((1, 1), jnp.float32),
    out_specs=pl.BlockSpec(memory_space=pltpu.SMEM),
)


def kernel(target, context, negative_samples, emb, ctx_emb):
    tgt = target.astype(jnp.int32)
    ctx = context.astype(jnp.int32)
    neg = negative_samples.astype(jnp.int32).reshape(_B * _K)
    comb = _tp_call(emb.T, ctx_emb.T)
    comb2 = comb.reshape(2 * _V, _D)
    pos_d, neg_d = _sc_dots()(tgt * 2, ctx * 2 + 1, neg * 2 + 1, comb2)
    loss = _loss_call(pos_d.reshape(_B // _DP, _DP),
                      neg_d.reshape(_B * _K // _DP, _DP))
    return loss[0, 0]
